# baseline (speedup 1.0000x reference)
import functools

import jax
import jax.numpy as jnp
from jax import lax
from jax.experimental import pallas as pl
from jax.experimental.pallas import tpu as pltpu
from jax.experimental.pallas import tpu_sc as plsc

N_TOKENS = 16384
HALF_DIM = 64
CHUNK = 128

_info = plsc.get_sparse_core_info()
NC, NS = _info.num_cores, _info.num_subcores
NW = NC * NS
B_PER_W = N_TOKENS // NW
N_CHUNKS = B_PER_W // CHUNK


def _make_kernel():
    mesh = plsc.VectorSubcoreMesh(core_axis_name="c", subcore_axis_name="s")

    @functools.partial(
        pl.kernel,
        mesh=mesh,
        out_type=jax.ShapeDtypeStruct((N_TOKENS, 2 * HALF_DIM), jnp.float32),
        scratch_types=[
            pltpu.VMEM((N_CHUNKS, CHUNK), jnp.int32),
            pltpu.VMEM((N_CHUNKS, CHUNK), jnp.int32),
            pltpu.VMEM((B_PER_W, HALF_DIM), jnp.float32),
            pltpu.VMEM((B_PER_W, HALF_DIM), jnp.float32),
            pltpu.SemaphoreType.DMA,
        ],
        compiler_params=pltpu.CompilerParams(use_tc_tiling_on_sc=False),
    )
    def k(word_idx_hbm, suff_idx_hbm, w_word_hbm, w_suff_hbm, out_hbm,
          idx_w, idx_s, rows_w, rows_s, sem):
        wid = lax.axis_index("s") * NC + lax.axis_index("c")
        base = wid * B_PER_W
        pltpu.sync_copy(word_idx_hbm.at[wid], idx_w)
        pltpu.sync_copy(suff_idx_hbm.at[wid], idx_s)
        gathers = []
        for j in range(N_CHUNKS):
            gathers.append(pltpu.async_copy(
                w_word_hbm.at[idx_w.at[j]],
                rows_w.at[pl.ds(j * CHUNK, CHUNK)], sem))
            gathers.append(pltpu.async_copy(
                w_suff_hbm.at[idx_s.at[j]],
                rows_s.at[pl.ds(j * CHUNK, CHUNK)], sem))
        for c in gathers:
            c.wait()
        pltpu.sync_copy(rows_w, out_hbm.at[pl.ds(base, B_PER_W), pl.ds(0, HALF_DIM)])
        pltpu.sync_copy(rows_s, out_hbm.at[pl.ds(base, B_PER_W), pl.ds(HALF_DIM, HALF_DIM)])

    return k


_sc_lookup = _make_kernel()


def kernel(word_idx, suff_idx, W_word, W_suff):
    wi = word_idx.astype(jnp.int32).reshape(NW, N_CHUNKS, CHUNK)
    si = suff_idx.astype(jnp.int32).reshape(NW, N_CHUNKS, CHUNK)
    return _sc_lookup(wi, si, W_word, W_suff)
